# baseline (speedup 1.0000x reference)
import functools

import jax
import jax.numpy as jnp
from jax import lax
from jax.experimental import pallas as pl
from jax.experimental.pallas import tpu as pltpu
from jax.experimental.pallas import tpu_sc as plsc

DIM = 64
TERSE_ID = 5
VERBOSE_ID = 7

NC = 2
NS = 16
NW = NC * NS
LANES = 16

C0 = 128
C1 = 72
NBUF = 4


def _build(n_b: int, n_l: int):
    assert n_b % NW == 0
    rows_w = n_b // NW
    nslots = 2 * rows_w
    assert nslots % NBUF == 0

    mesh = plsc.VectorSubcoreMesh(
        core_axis_name="c", subcore_axis_name="s", num_cores=NC, num_subcores=NS
    )

    @functools.partial(
        pl.kernel,
        out_type=jax.ShapeDtypeStruct((n_b, n_l, DIM), jnp.float32),
        mesh=mesh,
        compiler_params=pltpu.CompilerParams(use_tc_tiling_on_sc=False),
        scratch_types=[
            pltpu.VMEM((rows_w, n_l), jnp.int32),
            pltpu.VMEM((2, DIM), jnp.float32),
            [pltpu.VMEM((C0, DIM), jnp.float32) for _ in range(NBUF)],
            [pltpu.SemaphoreType.DMA for _ in range(NBUF)],
            [pltpu.SemaphoreType.DMA for _ in range(NBUF)],
        ],
    )
    def k(ids_hbm, table_hbm, sd_hbm, out_hbm, idx_v, sd_v, bufs, gsems, ssems):
        wid = lax.axis_index("s") * NC + lax.axis_index("c")
        base = wid * rows_w
        pltpu.sync_copy(ids_hbm.at[pl.ds(base, rows_w)], idx_v)
        pltpu.sync_copy(sd_hbm, sd_v)

        def chunk_of(b):
            return (0, C0) if b % 2 == 0 else (C0, C1)

        def fire_slot(t, b):
            off, n = chunk_of(b)
            r = t // 2
            pltpu.async_copy(
                table_hbm.at[idx_v.at[r, pl.ds(off, n)]],
                bufs[b].at[pl.ds(0, n)],
                gsems[b],
            )

        def drain_gather(b):
            _, n = chunk_of(b)
            pltpu.make_async_copy(
                table_hbm.at[pl.ds(0, n)], bufs[b].at[pl.ds(0, n)], gsems[b]
            ).wait()

        def store_slot(t, b):
            off, n = chunk_of(b)
            r = t // 2
            pltpu.async_copy(
                bufs[b].at[pl.ds(0, n)],
                out_hbm.at[base + r, pl.ds(off, n)],
                ssems[b],
            )

        def drain_store(b):
            off, n = chunk_of(b)
            pltpu.make_async_copy(
                bufs[b].at[pl.ds(0, n)], out_hbm.at[0, pl.ds(off, n)], ssems[b]
            ).wait()

        def process_slot(t, b):
            buf = bufs[b]
            off, n = chunk_of(b)
            r = t // 2
            n_full = n // LANES
            offs = [off + j * LANES for j in range(n_full)]
            tail = n - n_full * LANES
            if tail:
                offs.append(off + n - LANES)
            macc = jnp.zeros((LANES,), jnp.int32)
            for o in offs:
                v = idx_v[r, pl.ds(o, LANES)]
                m = (v == TERSE_ID) | (v == VERBOSE_ID)
                macc = macc | jnp.where(m, 1, 0)
            any_match = macc[0]
            for lane in range(1, LANES):
                any_match = any_match | macc[lane]

            @pl.when(any_match > 0)
            def _slow():
                def grp_body(jj, carry):
                    v = idx_v[r, pl.ds(off + jj * LANES, LANES)]
                    for ll in range(LANES):
                        s = v[ll]
                        is5 = s == TERSE_ID
                        is7 = s == VERBOSE_ID

                        @pl.when(is5 | is7)
                        def _(jj=jj, ll=ll, is5=is5):
                            row = jj * LANES + ll
                            for c in range(DIM // LANES):
                                sl = pl.ds(c * LANES, LANES)
                                d = jnp.where(is5, sd_v[0, sl], sd_v[1, sl])
                                buf[row, sl] = buf[row, sl] + d

                    return carry

                lax.fori_loop(0, n_full, grp_body, 0)

                if tail:
                    v = idx_v[r, pl.ds(off + n - LANES, LANES)]
                    for ll in range(LANES - tail, LANES):
                        s = v[ll]
                        is5 = s == TERSE_ID
                        is7 = s == VERBOSE_ID

                        @pl.when(is5 | is7)
                        def _(ll=ll, is5=is5):
                            row = n - LANES + ll
                            for c in range(DIM // LANES):
                                sl = pl.ds(c * LANES, LANES)
                                d = jnp.where(is5, sd_v[0, sl], sd_v[1, sl])
                                buf[row, sl] = buf[row, sl] + d

        fire_slot(0, 0)
        fire_slot(1, 1)

        def round_body(s_, carry):
            for b in range(NBUF):
                t = NBUF * s_ + b
                drain_gather(b)
                process_slot(t, b)
                store_slot(t, b)
                b2 = (b + 2) % NBUF

                @pl.when((t + 2 < nslots) & (t >= 2))
                def _(b2=b2):
                    drain_store(b2)

                @pl.when(t + 2 < nslots)
                def _(t=t, b2=b2):
                    fire_slot(t + 2, b2)

            return carry

        lax.fori_loop(0, nslots // NBUF, round_body, 0)
        for b in range(NBUF):
            drain_store(b)

    return k


_gather = _build(4096, 200)


@jax.jit
def kernel(input_ids, table, style_delta):
    return _gather(input_ids, table, style_delta)


# speedup vs baseline: 1.3586x; 1.3586x over previous
import functools

import jax
import jax.numpy as jnp
from jax import lax
from jax.experimental import pallas as pl
from jax.experimental.pallas import tpu as pltpu
from jax.experimental.pallas import tpu_sc as plsc

DIM = 64
TERSE_ID = 5
VERBOSE_ID = 7

NC = 2
NS = 16
NW = NC * NS
LANES = 16

CHUNK = 128
KG = 2
ROWS = KG * CHUNK
NBUF = 4


def _build(n_total: int):
    assert n_total % NW == 0
    n_w = n_total // NW
    assert n_w % ROWS == 0
    nslots = n_w // ROWS
    assert nslots % NBUF == 0

    mesh = plsc.VectorSubcoreMesh(
        core_axis_name="c", subcore_axis_name="s", num_cores=NC, num_subcores=NS
    )

    @functools.partial(
        pl.kernel,
        out_type=jax.ShapeDtypeStruct((n_total, 2 * DIM), jnp.float32),
        mesh=mesh,
        compiler_params=pltpu.CompilerParams(use_tc_tiling_on_sc=False),
        scratch_types=[
            pltpu.VMEM((n_w,), jnp.int32),
            pltpu.VMEM((2, DIM), jnp.float32),
            [pltpu.VMEM((ROWS, DIM), jnp.float32) for _ in range(NBUF)],
            [pltpu.SemaphoreType.DMA for _ in range(NBUF)],
            [pltpu.SemaphoreType.DMA for _ in range(NBUF)],
        ],
    )
    def k(ids_hbm, table_hbm, sd_hbm, out_hbm, idx_v, sd_v, bufs, gsems, ssems):
        wid = lax.axis_index("s") * NC + lax.axis_index("c")
        base = wid * n_w
        pltpu.sync_copy(ids_hbm.at[pl.ds(base, n_w)], idx_v)
        pltpu.sync_copy(sd_hbm, sd_v)

        def fire_slot(t, b):
            for kk in range(KG):
                pltpu.async_copy(
                    table_hbm.at[idx_v.at[pl.ds(t * ROWS + kk * CHUNK, CHUNK)]],
                    bufs[b].at[pl.ds(kk * CHUNK, CHUNK)],
                    gsems[b],
                )

        def drain_gather(b):
            pltpu.make_async_copy(
                table_hbm.at[pl.ds(0, ROWS)], bufs[b], gsems[b]
            ).wait()

        def store_slot(t, b):
            pltpu.async_copy(
                bufs[b],
                out_hbm.at[pl.ds(base + t * ROWS, ROWS), pl.ds(0, DIM)],
                ssems[b],
            )

        def drain_store(b):
            pltpu.make_async_copy(
                bufs[b], out_hbm.at[pl.ds(0, ROWS), pl.ds(0, DIM)], ssems[b]
            ).wait()

        def process_slot(t, b):
            buf = bufs[b]
            cb = t * ROWS
            macc = jnp.zeros((LANES,), jnp.int32)
            for j in range(ROWS // LANES):
                v = idx_v[pl.ds(cb + j * LANES, LANES)]
                m = (v == TERSE_ID) | (v == VERBOSE_ID)
                macc = macc | jnp.where(m, 1, 0)
            any_match = macc[0]
            for lane in range(1, LANES):
                any_match = any_match | macc[lane]

            @pl.when(any_match > 0)
            def _slow():
                def grp_body(jj, carry):
                    v = idx_v[pl.ds(cb + jj * LANES, LANES)]
                    for ll in range(LANES):
                        s = v[ll]
                        is5 = s == TERSE_ID
                        is7 = s == VERBOSE_ID

                        @pl.when(is5 | is7)
                        def _(jj=jj, ll=ll, is5=is5):
                            row = jj * LANES + ll
                            for c in range(DIM // LANES):
                                sl = pl.ds(c * LANES, LANES)
                                d = jnp.where(is5, sd_v[0, sl], sd_v[1, sl])
                                buf[row, sl] = buf[row, sl] + d

                    return carry

                lax.fori_loop(0, ROWS // LANES, grp_body, 0)

        fire_slot(0, 0)
        fire_slot(1, 1)

        def round_body(s_, carry):
            for b in range(NBUF):
                t = NBUF * s_ + b
                drain_gather(b)
                process_slot(t, b)
                store_slot(t, b)
                b2 = (b + 2) % NBUF

                @pl.when((t + 2 < nslots) & (t >= 2))
                def _(b2=b2):
                    drain_store(b2)

                @pl.when(t + 2 < nslots)
                def _(t=t, b2=b2):
                    fire_slot(t + 2, b2)

            return carry

        lax.fori_loop(0, nslots // NBUF, round_body, 0)
        for b in range(NBUF):
            drain_store(b)

    return k


_N_TOTAL = 4096 * 200
_gather = _build(_N_TOTAL)


@jax.jit
def kernel(input_ids, table, style_delta):
    b, l = input_ids.shape
    ids_flat = input_ids.reshape(-1)
    out = _gather(ids_flat, table, style_delta)
    return out[:, :DIM].reshape(b, l, DIM)
